# baseline (speedup 1.0000x reference)
import jax
import jax.numpy as jnp
from jax import lax
from jax.experimental import pallas as pl
from jax.experimental.pallas import tpu as pltpu
from jax.experimental.pallas import tpu_sc as plsc

N = 10000
D = 128
E = 320000

NS = 16
EPT = E // NS
CH = 32
NCHUNK = EPT // CH
ROWS_SUB = 624
TAIL = N - NS * ROWS_SUB

_MB = 400


def _mm_body(x_ref, w_ref, o_ref):
    o_ref[...] = jnp.dot(x_ref[...], w_ref[...], preferred_element_type=jnp.float32)


def _mm(x, w):
    return pl.pallas_call(
        _mm_body,
        grid=(N // _MB,),
        in_specs=[
            pl.BlockSpec((_MB, D), lambda i: (i, 0)),
            pl.BlockSpec((D, D), lambda i: (0, 0)),
        ],
        out_specs=pl.BlockSpec((_MB, D), lambda i: (i, 0)),
        out_shape=jax.ShapeDtypeStruct((N, D), jnp.float32),
    )(x, w)


def _proj_body(x_ref, w_ref, a_ref, hp_ref, lr_ref):
    hp = jnp.dot(x_ref[...], w_ref[...], preferred_element_type=jnp.float32)
    hp_ref[...] = hp
    lr_ref[...] = jnp.dot(hp, a_ref[...], preferred_element_type=jnp.float32)


def _proj(x, w, a2):
    return pl.pallas_call(
        _proj_body,
        grid=(N // _MB,),
        in_specs=[
            pl.BlockSpec((_MB, D), lambda i: (i, 0)),
            pl.BlockSpec((D, D), lambda i: (0, 0)),
            pl.BlockSpec((D, 8), lambda i: (0, 0)),
        ],
        out_specs=[
            pl.BlockSpec((_MB, D), lambda i: (i, 0)),
            pl.BlockSpec((_MB, 8), lambda i: (i, 0)),
        ],
        out_shape=[
            jax.ShapeDtypeStruct((N, D), jnp.float32),
            jax.ShapeDtypeStruct((N, 8), jnp.float32),
        ],
    )(x, w, a2)


def _combine_proj_body(n_ref, d_ref, w_ref, a_ref, hp_ref, lr_ref):
    den = d_ref[0, 0, :] + 1e-9
    x = n_ref[...] / den[:, None]
    x = jnp.where(x > 0, x, jnp.exp(x) - 1.0)
    hp = jnp.dot(x, w_ref[...], preferred_element_type=jnp.float32)
    hp_ref[...] = hp
    lr_ref[...] = jnp.dot(hp, a_ref[...], preferred_element_type=jnp.float32)


def _combine_proj(num, den, w, a2):
    den = den.reshape(N // _MB, 1, _MB)
    return pl.pallas_call(
        _combine_proj_body,
        grid=(N // _MB,),
        in_specs=[
            pl.BlockSpec((_MB, D), lambda i: (i, 0)),
            pl.BlockSpec((1, 1, _MB), lambda i: (i, 0, 0)),
            pl.BlockSpec((D, D), lambda i: (0, 0)),
            pl.BlockSpec((D, 8), lambda i: (0, 0)),
        ],
        out_specs=[
            pl.BlockSpec((_MB, D), lambda i: (i, 0)),
            pl.BlockSpec((_MB, 8), lambda i: (i, 0)),
        ],
        out_shape=[
            jax.ShapeDtypeStruct((N, D), jnp.float32),
            jax.ShapeDtypeStruct((N, 8), jnp.float32),
        ],
    )(num, den, w, a2)


def _combine_out_body(n_ref, d_ref, w_ref, o_ref):
    den = d_ref[0, 0, :] + 1e-9
    x = n_ref[...] / den[:, None]
    x = jnp.where(x > 0, x, jnp.exp(x) - 1.0)
    o_ref[...] = jnp.dot(x, w_ref[...], preferred_element_type=jnp.float32)


def _combine_out(num, den, w):
    den = den.reshape(N // _MB, 1, _MB)
    return pl.pallas_call(
        _combine_out_body,
        grid=(N // _MB,),
        in_specs=[
            pl.BlockSpec((_MB, D), lambda i: (i, 0)),
            pl.BlockSpec((1, 1, _MB), lambda i: (i, 0, 0)),
            pl.BlockSpec((D, D), lambda i: (0, 0)),
        ],
        out_specs=pl.BlockSpec((_MB, D), lambda i: (i, 0)),
        out_shape=jax.ShapeDtypeStruct((N, D), jnp.float32),
    )(num, den, w)


def _edge_body(hp, el, er, srcm, dstm, znd, zn, num_o, den_o,
               el_v, er_v, srcc, dstc, wc, rows_v, num_s, den_s,
               isem, gsem, ssem, dsem):
    s = lax.axis_index("s")

    pltpu.sync_copy(znd.at[pl.ds(s * ROWS_SUB, ROWS_SUB)],
                    num_s.at[pl.ds(s * ROWS_SUB, ROWS_SUB)])

    @pl.when(s == NS - 1)
    def _():
        pltpu.sync_copy(znd.at[pl.ds(NS * ROWS_SUB, TAIL)],
                        num_s.at[pl.ds(NS * ROWS_SUB, TAIL)])

    @pl.when(s == 0)
    def _():
        pltpu.sync_copy(zn, den_s)

    pltpu.sync_copy(el, el_v)
    pltpu.sync_copy(er, er_v)

    plsc.subcore_barrier()

    def idx_fetch(ci, b):
        pltpu.async_copy(srcm.at[s, ci], srcc.at[b], isem.at[b])
        pltpu.async_copy(dstm.at[s, ci], dstc.at[b], isem.at[b])

    def idx_wait(ci, b):
        pltpu.make_async_copy(srcm.at[s, ci], srcc.at[b], isem.at[b]).wait()
        pltpu.make_async_copy(dstm.at[s, ci], dstc.at[b], isem.at[b]).wait()

    def compute_w(b):
        for j in range(CH // 16):
            sl = pl.ds(j * 16, 16)
            s16 = srcc[b, sl]
            d16 = dstc[b, sl]
            x = plsc.load_gather(el_v, [s16]) + plsc.load_gather(er_v, [d16])
            e = jnp.maximum(x, 0.2 * x)
            wc[b, sl] = jnp.exp(e)

    def num_wait(cj):
        br = lax.rem(cj, 4)
        pltpu.make_async_copy(rows_v.at[br], num_s.at[dstc.at[lax.rem(cj, 8)]],
                              ssem.at[br]).wait()

    def den_wait(cj):
        b8 = lax.rem(cj, 8)
        pltpu.make_async_copy(wc.at[b8], den_s.at[dstc.at[b8]],
                              dsem.at[b8]).wait()

    def prep(ci):
        b8 = lax.rem(ci, 8)
        idx_wait(ci, b8)
        compute_w(b8)
        pltpu.async_copy(hp.at[srcc.at[b8]], rows_v.at[lax.rem(ci, 4)],
                         gsem.at[lax.rem(ci, 4)])

    for ci in range(4):
        idx_fetch(ci, ci)
    prep(0)
    prep(1)

    def _iter(ci, carry):
        br = lax.rem(ci, 4)
        b8 = lax.rem(ci, 8)

        @pl.when(ci >= 2)
        def _():
            num_wait(ci - 2)

        @pl.when(ci + 2 < NCHUNK)
        def _():
            prep(ci + 2)

        @pl.when(ci + 4 < NCHUNK)
        def _():
            @pl.when(ci >= 4)
            def _():
                den_wait(ci - 4)

            idx_fetch(ci + 4, lax.rem(ci + 4, 8))

        pltpu.make_async_copy(hp.at[srcc.at[b8]], rows_v.at[br],
                              gsem.at[br]).wait()

        for g in range(CH // 16):
            w16 = wc[b8, pl.ds(g * 16, 16)]
            for k in range(16):
                w1 = w16[k]
                for j in range(D // 16):
                    sl = pl.ds(j * 16, 16)
                    i = g * 16 + k
                    rows_v[br, i, sl] = rows_v[br, i, sl] * w1

        pltpu.async_copy(wc.at[b8], den_s.at[dstc.at[b8]], dsem.at[b8],
                         add=True)
        pltpu.async_copy(rows_v.at[br], num_s.at[dstc.at[b8]], ssem.at[br],
                         add=True)
        return carry

    lax.fori_loop(0, NCHUNK, _iter, 0)

    num_wait(NCHUNK - 2)
    num_wait(NCHUNK - 1)
    for k in range(8):
        den_wait(NCHUNK - 8 + k)

    plsc.subcore_barrier()

    pltpu.sync_copy(num_s.at[pl.ds(s * ROWS_SUB, ROWS_SUB)],
                    num_o.at[pl.ds(s * ROWS_SUB, ROWS_SUB)])

    @pl.when(s == NS - 1)
    def _():
        pltpu.sync_copy(num_s.at[pl.ds(NS * ROWS_SUB, TAIL)],
                        num_o.at[pl.ds(NS * ROWS_SUB, TAIL)])

    @pl.when(s == 0)
    def _():
        pltpu.sync_copy(den_s, den_o)


_edge = pl.kernel(
    _edge_body,
    out_type=[
        jax.ShapeDtypeStruct((N, D), jnp.float32),
        jax.ShapeDtypeStruct((N,), jnp.float32),
    ],
    mesh=plsc.VectorSubcoreMesh(core_axis_name="c", subcore_axis_name="s",
                                num_cores=1, num_subcores=NS),
    compiler_params=pltpu.CompilerParams(needs_layout_passes=False),
    scratch_types=[
        pltpu.VMEM((N,), jnp.float32),
        pltpu.VMEM((N,), jnp.float32),
        pltpu.VMEM((8, CH), jnp.int32),
        pltpu.VMEM((8, CH), jnp.int32),
        pltpu.VMEM((8, CH), jnp.float32),
        pltpu.VMEM((4, CH, D), jnp.float32),
        pltpu.VMEM_SHARED((N, D), jnp.float32),
        pltpu.VMEM_SHARED((N,), jnp.float32),
        pltpu.SemaphoreType.DMA((8,)),
        pltpu.SemaphoreType.DMA((4,)),
        pltpu.SemaphoreType.DMA((4,)),
        pltpu.SemaphoreType.DMA((8,)),
    ],
)


def _pack_a(a_l, a_r):
    a2 = jnp.zeros((D, 8), jnp.float32)
    return a2.at[:, 0].set(a_l).at[:, 1].set(a_r)


def kernel(h, edge_index, W_in, W1, a_l1, a_r1, W2, a_l2, a_r2, W_out):
    srcm = edge_index[0].reshape(NS, NCHUNK, CH)
    dstm = edge_index[1].reshape(NS, NCHUNK, CH)
    znd = jnp.zeros((N, D), jnp.float32)
    zn = jnp.zeros((N,), jnp.float32)

    x0 = _mm(h, W_in)

    hp1, lr1 = _proj(x0, W1, _pack_a(a_l1, a_r1))
    num1, den1 = _edge(hp1, lr1[:, 0], lr1[:, 1], srcm, dstm, znd, zn)

    hp2, lr2 = _combine_proj(num1, den1, W2, _pack_a(a_l2, a_r2))
    num2, den2 = _edge(hp2, lr2[:, 0], lr2[:, 1], srcm, dstm, znd, zn)

    return _combine_out(num2, den2, W_out)


# speedup vs baseline: 1.0212x; 1.0212x over previous
import jax
import jax.numpy as jnp
from jax import lax
from jax.experimental import pallas as pl
from jax.experimental.pallas import tpu as pltpu
from jax.experimental.pallas import tpu_sc as plsc

N = 10000
D = 128
E = 320000

NS = 16
EPT = E // NS
CH = 32
NCHUNK = EPT // CH
ROWS_SUB = 624
TAIL = N - NS * ROWS_SUB

_MB = 400


def _mm_body(x_ref, w_ref, o_ref):
    o_ref[...] = jnp.dot(x_ref[...], w_ref[...], preferred_element_type=jnp.float32)


def _mm(x, w):
    return pl.pallas_call(
        _mm_body,
        grid=(N // _MB,),
        in_specs=[
            pl.BlockSpec((_MB, D), lambda i: (i, 0)),
            pl.BlockSpec((D, D), lambda i: (0, 0)),
        ],
        out_specs=pl.BlockSpec((_MB, D), lambda i: (i, 0)),
        out_shape=jax.ShapeDtypeStruct((N, D), jnp.float32),
    )(x, w)


def _proj_body(x_ref, w0_ref, w_ref, a_ref, hp_ref, lr_ref):
    x0 = jnp.dot(x_ref[...], w0_ref[...], preferred_element_type=jnp.float32)
    hp = jnp.dot(x0, w_ref[...], preferred_element_type=jnp.float32)
    hp_ref[...] = hp
    lr_ref[...] = jnp.dot(hp, a_ref[...], preferred_element_type=jnp.float32)


def _proj(x, w0, w, a2):
    return pl.pallas_call(
        _proj_body,
        grid=(N // _MB,),
        in_specs=[
            pl.BlockSpec((_MB, D), lambda i: (i, 0)),
            pl.BlockSpec((D, D), lambda i: (0, 0)),
            pl.BlockSpec((D, D), lambda i: (0, 0)),
            pl.BlockSpec((D, 8), lambda i: (0, 0)),
        ],
        out_specs=[
            pl.BlockSpec((_MB, D), lambda i: (i, 0)),
            pl.BlockSpec((_MB, 8), lambda i: (i, 0)),
        ],
        out_shape=[
            jax.ShapeDtypeStruct((N, D), jnp.float32),
            jax.ShapeDtypeStruct((N, 8), jnp.float32),
        ],
    )(x, w0, w, a2)


def _combine_proj_body(n_ref, d_ref, w_ref, a_ref, hp_ref, lr_ref):
    den = d_ref[0, 0, :] + 1e-9
    x = n_ref[...] / den[:, None]
    x = jnp.where(x > 0, x, jnp.exp(x) - 1.0)
    hp = jnp.dot(x, w_ref[...], preferred_element_type=jnp.float32)
    hp_ref[...] = hp
    lr_ref[...] = jnp.dot(hp, a_ref[...], preferred_element_type=jnp.float32)


def _combine_proj(num, den, w, a2):
    den = den.reshape(N // _MB, 1, _MB)
    return pl.pallas_call(
        _combine_proj_body,
        grid=(N // _MB,),
        in_specs=[
            pl.BlockSpec((_MB, D), lambda i: (i, 0)),
            pl.BlockSpec((1, 1, _MB), lambda i: (i, 0, 0)),
            pl.BlockSpec((D, D), lambda i: (0, 0)),
            pl.BlockSpec((D, 8), lambda i: (0, 0)),
        ],
        out_specs=[
            pl.BlockSpec((_MB, D), lambda i: (i, 0)),
            pl.BlockSpec((_MB, 8), lambda i: (i, 0)),
        ],
        out_shape=[
            jax.ShapeDtypeStruct((N, D), jnp.float32),
            jax.ShapeDtypeStruct((N, 8), jnp.float32),
        ],
    )(num, den, w, a2)


def _combine_out_body(n_ref, d_ref, w_ref, o_ref):
    den = d_ref[0, 0, :] + 1e-9
    x = n_ref[...] / den[:, None]
    x = jnp.where(x > 0, x, jnp.exp(x) - 1.0)
    o_ref[...] = jnp.dot(x, w_ref[...], preferred_element_type=jnp.float32)


def _combine_out(num, den, w):
    den = den.reshape(N // _MB, 1, _MB)
    return pl.pallas_call(
        _combine_out_body,
        grid=(N // _MB,),
        in_specs=[
            pl.BlockSpec((_MB, D), lambda i: (i, 0)),
            pl.BlockSpec((1, 1, _MB), lambda i: (i, 0, 0)),
            pl.BlockSpec((D, D), lambda i: (0, 0)),
        ],
        out_specs=pl.BlockSpec((_MB, D), lambda i: (i, 0)),
        out_shape=jax.ShapeDtypeStruct((N, D), jnp.float32),
    )(num, den, w)


def _edge_body(hp, el, er, srcm, dstm, znd, zn, num_o, den_o,
               el_v, er_v, srcc, dstc, wc, rows_v, num_s, den_s,
               isem, gsem, ssem, dsem):
    s = lax.axis_index("s")

    pltpu.sync_copy(znd.at[pl.ds(s * ROWS_SUB, ROWS_SUB)],
                    num_s.at[pl.ds(s * ROWS_SUB, ROWS_SUB)])

    @pl.when(s == NS - 1)
    def _():
        pltpu.sync_copy(znd.at[pl.ds(NS * ROWS_SUB, TAIL)],
                        num_s.at[pl.ds(NS * ROWS_SUB, TAIL)])

    @pl.when(s == 0)
    def _():
        pltpu.sync_copy(zn, den_s)

    pltpu.sync_copy(el, el_v)
    pltpu.sync_copy(er, er_v)

    plsc.subcore_barrier()

    def idx_fetch(ci, b):
        pltpu.async_copy(srcm.at[s, ci], srcc.at[b], isem.at[b])
        pltpu.async_copy(dstm.at[s, ci], dstc.at[b], isem.at[b])

    def idx_wait(ci, b):
        pltpu.make_async_copy(srcm.at[s, ci], srcc.at[b], isem.at[b]).wait()
        pltpu.make_async_copy(dstm.at[s, ci], dstc.at[b], isem.at[b]).wait()

    def compute_w(b):
        for j in range(CH // 16):
            sl = pl.ds(j * 16, 16)
            s16 = srcc[b, sl]
            d16 = dstc[b, sl]
            x = plsc.load_gather(el_v, [s16]) + plsc.load_gather(er_v, [d16])
            e = jnp.maximum(x, 0.2 * x)
            wc[b, sl] = jnp.exp(e)

    def num_wait(cj):
        br = lax.rem(cj, 6)
        pltpu.make_async_copy(rows_v.at[br], num_s.at[dstc.at[lax.rem(cj, 8)]],
                              ssem.at[br]).wait()

    def den_wait(cj):
        b8 = lax.rem(cj, 8)
        pltpu.make_async_copy(wc.at[b8], den_s.at[dstc.at[b8]],
                              dsem.at[b8]).wait()

    def prep(ci):
        b8 = lax.rem(ci, 8)
        idx_wait(ci, b8)
        compute_w(b8)
        pltpu.async_copy(hp.at[srcc.at[b8]], rows_v.at[lax.rem(ci, 6)],
                         gsem.at[lax.rem(ci, 6)])

    for ci in range(4):
        idx_fetch(ci, ci)
    prep(0)
    prep(1)

    def _iter(ci, carry):
        br = lax.rem(ci, 6)
        b8 = lax.rem(ci, 8)

        @pl.when(ci >= 3)
        def _():
            num_wait(ci - 3)

        @pl.when(ci + 2 < NCHUNK)
        def _():
            prep(ci + 2)

        @pl.when(ci + 4 < NCHUNK)
        def _():
            @pl.when(ci >= 4)
            def _():
                den_wait(ci - 4)

            idx_fetch(ci + 4, lax.rem(ci + 4, 8))

        pltpu.make_async_copy(hp.at[srcc.at[b8]], rows_v.at[br],
                              gsem.at[br]).wait()

        for g in range(CH // 16):
            w16 = wc[b8, pl.ds(g * 16, 16)]
            for k in range(16):
                w1 = w16[k]
                for j in range(D // 16):
                    sl = pl.ds(j * 16, 16)
                    i = g * 16 + k
                    rows_v[br, i, sl] = rows_v[br, i, sl] * w1

        pltpu.async_copy(wc.at[b8], den_s.at[dstc.at[b8]], dsem.at[b8],
                         add=True)
        pltpu.async_copy(rows_v.at[br], num_s.at[dstc.at[b8]], ssem.at[br],
                         add=True)
        return carry

    lax.fori_loop(0, NCHUNK, _iter, 0)

    num_wait(NCHUNK - 3)
    num_wait(NCHUNK - 2)
    num_wait(NCHUNK - 1)
    for k in range(8):
        den_wait(NCHUNK - 8 + k)

    plsc.subcore_barrier()

    pltpu.sync_copy(num_s.at[pl.ds(s * ROWS_SUB, ROWS_SUB)],
                    num_o.at[pl.ds(s * ROWS_SUB, ROWS_SUB)])

    @pl.when(s == NS - 1)
    def _():
        pltpu.sync_copy(num_s.at[pl.ds(NS * ROWS_SUB, TAIL)],
                        num_o.at[pl.ds(NS * ROWS_SUB, TAIL)])

    @pl.when(s == 0)
    def _():
        pltpu.sync_copy(den_s, den_o)


_edge = pl.kernel(
    _edge_body,
    out_type=[
        jax.ShapeDtypeStruct((N, D), jnp.float32),
        jax.ShapeDtypeStruct((N,), jnp.float32),
    ],
    mesh=plsc.VectorSubcoreMesh(core_axis_name="c", subcore_axis_name="s",
                                num_cores=1, num_subcores=NS),
    compiler_params=pltpu.CompilerParams(needs_layout_passes=False),
    scratch_types=[
        pltpu.VMEM((N,), jnp.float32),
        pltpu.VMEM((N,), jnp.float32),
        pltpu.VMEM((8, CH), jnp.int32),
        pltpu.VMEM((8, CH), jnp.int32),
        pltpu.VMEM((8, CH), jnp.float32),
        pltpu.VMEM((6, CH, D), jnp.float32),
        pltpu.VMEM_SHARED((N, D), jnp.float32),
        pltpu.VMEM_SHARED((N,), jnp.float32),
        pltpu.SemaphoreType.DMA((8,)),
        pltpu.SemaphoreType.DMA((6,)),
        pltpu.SemaphoreType.DMA((6,)),
        pltpu.SemaphoreType.DMA((8,)),
    ],
)


def _pack_a(a_l, a_r):
    a2 = jnp.zeros((D, 8), jnp.float32)
    return a2.at[:, 0].set(a_l).at[:, 1].set(a_r)


def kernel(h, edge_index, W_in, W1, a_l1, a_r1, W2, a_l2, a_r2, W_out):
    srcm = edge_index[0].reshape(NS, NCHUNK, CH)
    dstm = edge_index[1].reshape(NS, NCHUNK, CH)
    znd = jnp.zeros((N, D), jnp.float32)
    zn = jnp.zeros((N,), jnp.float32)

    hp1, lr1 = _proj(h, W_in, W1, _pack_a(a_l1, a_r1))
    num1, den1 = _edge(hp1, lr1[:, 0], lr1[:, 1], srcm, dstm, znd, zn)

    hp2, lr2 = _combine_proj(num1, den1, W2, _pack_a(a_l2, a_r2))
    num2, den2 = _edge(hp2, lr2[:, 0], lr2[:, 1], srcm, dstm, znd, zn)

    return _combine_out(num2, den2, W_out)


# speedup vs baseline: 1.0427x; 1.0211x over previous
import jax
import jax.numpy as jnp
from jax import lax
from jax.experimental import pallas as pl
from jax.experimental.pallas import tpu as pltpu
from jax.experimental.pallas import tpu_sc as plsc

N = 10000
D = 128
E = 320000

NS = 16
EPT = E // NS
CH = 32
NCHUNK = EPT // CH
ROWS_SUB = 624
TAIL = N - NS * ROWS_SUB

_MB = 400


def _mm_body(x_ref, w_ref, o_ref):
    o_ref[...] = jnp.dot(x_ref[...], w_ref[...], preferred_element_type=jnp.float32)


def _mm(x, w):
    return pl.pallas_call(
        _mm_body,
        grid=(N // _MB,),
        in_specs=[
            pl.BlockSpec((_MB, D), lambda i: (i, 0)),
            pl.BlockSpec((D, D), lambda i: (0, 0)),
        ],
        out_specs=pl.BlockSpec((_MB, D), lambda i: (i, 0)),
        out_shape=jax.ShapeDtypeStruct((N, D), jnp.float32),
    )(x, w)


def _proj_body(x_ref, w0_ref, w_ref, al_ref, ar_ref, hp_ref, el_ref, er_ref):
    x0 = jnp.dot(x_ref[...], w0_ref[...], preferred_element_type=jnp.float32)
    hp = jnp.dot(x0, w_ref[...], preferred_element_type=jnp.float32)
    hp_ref[...] = hp
    a2 = jnp.concatenate([al_ref[...][:, None], ar_ref[...][:, None]], axis=1)
    lr = jnp.dot(hp, a2, preferred_element_type=jnp.float32)
    el_ref[...] = lr[:, 0].reshape(1, 1, _MB)
    er_ref[...] = lr[:, 1].reshape(1, 1, _MB)


def _proj(x, w0, w, a_l, a_r):
    hp, el, er = pl.pallas_call(
        _proj_body,
        grid=(N // _MB,),
        in_specs=[
            pl.BlockSpec((_MB, D), lambda i: (i, 0)),
            pl.BlockSpec((D, D), lambda i: (0, 0)),
            pl.BlockSpec((D, D), lambda i: (0, 0)),
            pl.BlockSpec((D,), lambda i: (0,)),
            pl.BlockSpec((D,), lambda i: (0,)),
        ],
        out_specs=[
            pl.BlockSpec((_MB, D), lambda i: (i, 0)),
            pl.BlockSpec((1, 1, _MB), lambda i: (i, 0, 0)),
            pl.BlockSpec((1, 1, _MB), lambda i: (i, 0, 0)),
        ],
        out_shape=[
            jax.ShapeDtypeStruct((N, D), jnp.float32),
            jax.ShapeDtypeStruct((N // _MB, 1, _MB), jnp.float32),
            jax.ShapeDtypeStruct((N // _MB, 1, _MB), jnp.float32),
        ],
    )(x, w0, w, a_l, a_r)
    return hp, el.reshape(N), er.reshape(N)


def _combine_proj_body(n_ref, d_ref, w_ref, al_ref, ar_ref, hp_ref, el_ref,
                       er_ref):
    den = d_ref[0, 0, :] + 1e-9
    x = n_ref[...] / den[:, None]
    x = jnp.where(x > 0, x, jnp.exp(x) - 1.0)
    hp = jnp.dot(x, w_ref[...], preferred_element_type=jnp.float32)
    hp_ref[...] = hp
    a2 = jnp.concatenate([al_ref[...][:, None], ar_ref[...][:, None]], axis=1)
    lr = jnp.dot(hp, a2, preferred_element_type=jnp.float32)
    el_ref[...] = lr[:, 0].reshape(1, 1, _MB)
    er_ref[...] = lr[:, 1].reshape(1, 1, _MB)


def _combine_proj(num, den, w, a_l, a_r):
    den = den.reshape(N // _MB, 1, _MB)
    hp, el, er = pl.pallas_call(
        _combine_proj_body,
        grid=(N // _MB,),
        in_specs=[
            pl.BlockSpec((_MB, D), lambda i: (i, 0)),
            pl.BlockSpec((1, 1, _MB), lambda i: (i, 0, 0)),
            pl.BlockSpec((D, D), lambda i: (0, 0)),
            pl.BlockSpec((D,), lambda i: (0,)),
            pl.BlockSpec((D,), lambda i: (0,)),
        ],
        out_specs=[
            pl.BlockSpec((_MB, D), lambda i: (i, 0)),
            pl.BlockSpec((1, 1, _MB), lambda i: (i, 0, 0)),
            pl.BlockSpec((1, 1, _MB), lambda i: (i, 0, 0)),
        ],
        out_shape=[
            jax.ShapeDtypeStruct((N, D), jnp.float32),
            jax.ShapeDtypeStruct((N // _MB, 1, _MB), jnp.float32),
            jax.ShapeDtypeStruct((N // _MB, 1, _MB), jnp.float32),
        ],
    )(num, den, w, a_l, a_r)
    return hp, el.reshape(N), er.reshape(N)


def _combine_out_body(n_ref, d_ref, w_ref, o_ref):
    den = d_ref[0, 0, :] + 1e-9
    x = n_ref[...] / den[:, None]
    x = jnp.where(x > 0, x, jnp.exp(x) - 1.0)
    o_ref[...] = jnp.dot(x, w_ref[...], preferred_element_type=jnp.float32)


def _combine_out(num, den, w):
    den = den.reshape(N // _MB, 1, _MB)
    return pl.pallas_call(
        _combine_out_body,
        grid=(N // _MB,),
        in_specs=[
            pl.BlockSpec((_MB, D), lambda i: (i, 0)),
            pl.BlockSpec((1, 1, _MB), lambda i: (i, 0, 0)),
            pl.BlockSpec((D, D), lambda i: (0, 0)),
        ],
        out_specs=pl.BlockSpec((_MB, D), lambda i: (i, 0)),
        out_shape=jax.ShapeDtypeStruct((N, D), jnp.float32),
    )(num, den, w)


def _edge_body(hp, el, er, srcm, dstm, znd, zn, num_o, den_o,
               el_v, er_v, srcc, dstc, wc, rows_v, num_s, den_s,
               isem, gsem, ssem, dsem):
    s = lax.axis_index("s")

    pltpu.sync_copy(znd.at[pl.ds(s * ROWS_SUB, ROWS_SUB)],
                    num_s.at[pl.ds(s * ROWS_SUB, ROWS_SUB)])

    @pl.when(s == NS - 1)
    def _():
        pltpu.sync_copy(znd.at[pl.ds(NS * ROWS_SUB, TAIL)],
                        num_s.at[pl.ds(NS * ROWS_SUB, TAIL)])

    @pl.when(s == 0)
    def _():
        pltpu.sync_copy(zn, den_s)

    pltpu.sync_copy(el, el_v)
    pltpu.sync_copy(er, er_v)

    plsc.subcore_barrier()

    def idx_fetch(ci, b):
        pltpu.async_copy(srcm.at[s, ci], srcc.at[b], isem.at[b])
        pltpu.async_copy(dstm.at[s, ci], dstc.at[b], isem.at[b])

    def idx_wait(ci, b):
        pltpu.make_async_copy(srcm.at[s, ci], srcc.at[b], isem.at[b]).wait()
        pltpu.make_async_copy(dstm.at[s, ci], dstc.at[b], isem.at[b]).wait()

    def compute_w(b):
        for j in range(CH // 16):
            sl = pl.ds(j * 16, 16)
            s16 = srcc[b, sl]
            d16 = dstc[b, sl]
            x = plsc.load_gather(el_v, [s16]) + plsc.load_gather(er_v, [d16])
            e = jnp.maximum(x, 0.2 * x)
            wc[b, sl] = jnp.exp(e)

    def num_wait(cj):
        br = lax.rem(cj, 6)
        pltpu.make_async_copy(rows_v.at[br], num_s.at[dstc.at[lax.rem(cj, 8)]],
                              ssem.at[br]).wait()

    def den_wait(cj):
        b8 = lax.rem(cj, 8)
        pltpu.make_async_copy(wc.at[b8], den_s.at[dstc.at[b8]],
                              dsem.at[b8]).wait()

    def prep(ci):
        b8 = lax.rem(ci, 8)
        idx_wait(ci, b8)
        compute_w(b8)
        pltpu.async_copy(hp.at[srcc.at[b8]], rows_v.at[lax.rem(ci, 6)],
                         gsem.at[lax.rem(ci, 6)])

    for ci in range(4):
        idx_fetch(ci, ci)
    prep(0)
    prep(1)

    def _iter(ci, carry):
        br = lax.rem(ci, 6)
        b8 = lax.rem(ci, 8)

        @pl.when(ci >= 3)
        def _():
            num_wait(ci - 3)

        @pl.when(ci + 2 < NCHUNK)
        def _():
            prep(ci + 2)

        @pl.when(ci + 4 < NCHUNK)
        def _():
            @pl.when(ci >= 4)
            def _():
                den_wait(ci - 4)

            idx_fetch(ci + 4, lax.rem(ci + 4, 8))

        pltpu.make_async_copy(hp.at[srcc.at[b8]], rows_v.at[br],
                              gsem.at[br]).wait()

        for g in range(CH // 16):
            w16 = wc[b8, pl.ds(g * 16, 16)]
            for k in range(16):
                w1 = w16[k]
                for j in range(D // 16):
                    sl = pl.ds(j * 16, 16)
                    i = g * 16 + k
                    rows_v[br, i, sl] = rows_v[br, i, sl] * w1

        pltpu.async_copy(wc.at[b8], den_s.at[dstc.at[b8]], dsem.at[b8],
                         add=True)
        pltpu.async_copy(rows_v.at[br], num_s.at[dstc.at[b8]], ssem.at[br],
                         add=True)
        return carry

    lax.fori_loop(0, NCHUNK, _iter, 0)

    num_wait(NCHUNK - 3)
    num_wait(NCHUNK - 2)
    num_wait(NCHUNK - 1)
    for k in range(8):
        den_wait(NCHUNK - 8 + k)

    plsc.subcore_barrier()

    pltpu.sync_copy(num_s.at[pl.ds(s * ROWS_SUB, ROWS_SUB)],
                    num_o.at[pl.ds(s * ROWS_SUB, ROWS_SUB)])

    @pl.when(s == NS - 1)
    def _():
        pltpu.sync_copy(num_s.at[pl.ds(NS * ROWS_SUB, TAIL)],
                        num_o.at[pl.ds(NS * ROWS_SUB, TAIL)])

    @pl.when(s == 0)
    def _():
        pltpu.sync_copy(den_s, den_o)


_edge = pl.kernel(
    _edge_body,
    out_type=[
        jax.ShapeDtypeStruct((N, D), jnp.float32),
        jax.ShapeDtypeStruct((N,), jnp.float32),
    ],
    mesh=plsc.VectorSubcoreMesh(core_axis_name="c", subcore_axis_name="s",
                                num_cores=1, num_subcores=NS),
    compiler_params=pltpu.CompilerParams(needs_layout_passes=False),
    scratch_types=[
        pltpu.VMEM((N,), jnp.float32),
        pltpu.VMEM((N,), jnp.float32),
        pltpu.VMEM((8, CH), jnp.int32),
        pltpu.VMEM((8, CH), jnp.int32),
        pltpu.VMEM((8, CH), jnp.float32),
        pltpu.VMEM((6, CH, D), jnp.float32),
        pltpu.VMEM_SHARED((N, D), jnp.float32),
        pltpu.VMEM_SHARED((N,), jnp.float32),
        pltpu.SemaphoreType.DMA((8,)),
        pltpu.SemaphoreType.DMA((6,)),
        pltpu.SemaphoreType.DMA((6,)),
        pltpu.SemaphoreType.DMA((8,)),
    ],
)


def kernel(h, edge_index, W_in, W1, a_l1, a_r1, W2, a_l2, a_r2, W_out):
    srcm = edge_index[0].reshape(NS, NCHUNK, CH)
    dstm = edge_index[1].reshape(NS, NCHUNK, CH)
    znd = jnp.zeros((N, D), jnp.float32)
    zn = jnp.zeros((N,), jnp.float32)

    hp1, el1, er1 = _proj(h, W_in, W1, a_l1, a_r1)
    num1, den1 = _edge(hp1, el1, er1, srcm, dstm, znd, zn)

    hp2, el2, er2 = _combine_proj(num1, den1, W2, a_l2, a_r2)
    num2, den2 = _edge(hp2, el2, er2, srcm, dstm, znd, zn)

    return _combine_out(num2, den2, W_out)


# speedup vs baseline: 1.0435x; 1.0007x over previous
import jax
import jax.numpy as jnp
from jax import lax
from jax.experimental import pallas as pl
from jax.experimental.pallas import tpu as pltpu
from jax.experimental.pallas import tpu_sc as plsc

N = 10000
D = 128
E = 320000

NS = 16
EPT = E // NS
CH = 32
NCHUNK = EPT // CH
ROWS_SUB = 624
TAIL = N - NS * ROWS_SUB

_MB = 400


def _mm_body(x_ref, w_ref, o_ref):
    o_ref[...] = jnp.dot(x_ref[...], w_ref[...], preferred_element_type=jnp.float32)


def _mm(x, w):
    return pl.pallas_call(
        _mm_body,
        grid=(N // _MB,),
        in_specs=[
            pl.BlockSpec((_MB, D), lambda i: (i, 0)),
            pl.BlockSpec((D, D), lambda i: (0, 0)),
        ],
        out_specs=pl.BlockSpec((_MB, D), lambda i: (i, 0)),
        out_shape=jax.ShapeDtypeStruct((N, D), jnp.float32),
    )(x, w)


def _proj_body(x_ref, w0_ref, w_ref, al_ref, ar_ref, hp_ref, el_ref, er_ref):
    x0 = jnp.dot(x_ref[...], w0_ref[...], preferred_element_type=jnp.float32)
    hp = jnp.dot(x0, w_ref[...], preferred_element_type=jnp.float32)
    hp_ref[...] = hp
    a2 = jnp.concatenate([al_ref[...][:, None], ar_ref[...][:, None]], axis=1)
    lr = jnp.dot(hp, a2, preferred_element_type=jnp.float32)
    el_ref[...] = lr[:, 0].reshape(1, 1, _MB)
    er_ref[...] = lr[:, 1].reshape(1, 1, _MB)


def _proj(x, w0, w, a_l, a_r):
    hp, el, er = pl.pallas_call(
        _proj_body,
        grid=(N // _MB,),
        in_specs=[
            pl.BlockSpec((_MB, D), lambda i: (i, 0)),
            pl.BlockSpec((D, D), lambda i: (0, 0)),
            pl.BlockSpec((D, D), lambda i: (0, 0)),
            pl.BlockSpec((D,), lambda i: (0,)),
            pl.BlockSpec((D,), lambda i: (0,)),
        ],
        out_specs=[
            pl.BlockSpec((_MB, D), lambda i: (i, 0)),
            pl.BlockSpec((1, 1, _MB), lambda i: (i, 0, 0)),
            pl.BlockSpec((1, 1, _MB), lambda i: (i, 0, 0)),
        ],
        out_shape=[
            jax.ShapeDtypeStruct((N, D), jnp.float32),
            jax.ShapeDtypeStruct((N // _MB, 1, _MB), jnp.float32),
            jax.ShapeDtypeStruct((N // _MB, 1, _MB), jnp.float32),
        ],
    )(x, w0, w, a_l, a_r)
    return hp, el.reshape(N), er.reshape(N)


def _combine_proj_body(n_ref, d_ref, w_ref, al_ref, ar_ref, hp_ref, el_ref,
                       er_ref):
    den = d_ref[0, 0, :] + 1e-9
    x = n_ref[...] / den[:, None]
    x = jnp.where(x > 0, x, jnp.exp(x) - 1.0)
    hp = jnp.dot(x, w_ref[...], preferred_element_type=jnp.float32)
    hp_ref[...] = hp
    a2 = jnp.concatenate([al_ref[...][:, None], ar_ref[...][:, None]], axis=1)
    lr = jnp.dot(hp, a2, preferred_element_type=jnp.float32)
    el_ref[...] = lr[:, 0].reshape(1, 1, _MB)
    er_ref[...] = lr[:, 1].reshape(1, 1, _MB)


def _combine_proj(num, den, w, a_l, a_r):
    den = den.reshape(N // _MB, 1, _MB)
    hp, el, er = pl.pallas_call(
        _combine_proj_body,
        grid=(N // _MB,),
        in_specs=[
            pl.BlockSpec((_MB, D), lambda i: (i, 0)),
            pl.BlockSpec((1, 1, _MB), lambda i: (i, 0, 0)),
            pl.BlockSpec((D, D), lambda i: (0, 0)),
            pl.BlockSpec((D,), lambda i: (0,)),
            pl.BlockSpec((D,), lambda i: (0,)),
        ],
        out_specs=[
            pl.BlockSpec((_MB, D), lambda i: (i, 0)),
            pl.BlockSpec((1, 1, _MB), lambda i: (i, 0, 0)),
            pl.BlockSpec((1, 1, _MB), lambda i: (i, 0, 0)),
        ],
        out_shape=[
            jax.ShapeDtypeStruct((N, D), jnp.float32),
            jax.ShapeDtypeStruct((N // _MB, 1, _MB), jnp.float32),
            jax.ShapeDtypeStruct((N // _MB, 1, _MB), jnp.float32),
        ],
    )(num, den, w, a_l, a_r)
    return hp, el.reshape(N), er.reshape(N)


def _combine_out_body(n_ref, d_ref, w_ref, o_ref):
    den = d_ref[0, 0, :] + 1e-9
    x = n_ref[...] / den[:, None]
    x = jnp.where(x > 0, x, jnp.exp(x) - 1.0)
    o_ref[...] = jnp.dot(x, w_ref[...], preferred_element_type=jnp.float32)


def _combine_out(num, den, w):
    den = den.reshape(N // _MB, 1, _MB)
    return pl.pallas_call(
        _combine_out_body,
        grid=(N // _MB,),
        in_specs=[
            pl.BlockSpec((_MB, D), lambda i: (i, 0)),
            pl.BlockSpec((1, 1, _MB), lambda i: (i, 0, 0)),
            pl.BlockSpec((D, D), lambda i: (0, 0)),
        ],
        out_specs=pl.BlockSpec((_MB, D), lambda i: (i, 0)),
        out_shape=jax.ShapeDtypeStruct((N, D), jnp.float32),
    )(num, den, w)


def _edge_body(hp, el, er, srcm, dstm, znd, zn, num_o, den_o,
               el_v, er_v, srcc, dstc, wc, rows_v, num_s, den_s,
               isem, gsem, ssem, dsem):
    s = lax.axis_index("s")

    pltpu.sync_copy(znd.at[pl.ds(s * ROWS_SUB, ROWS_SUB)],
                    num_s.at[pl.ds(s * ROWS_SUB, ROWS_SUB)])

    @pl.when(s == NS - 1)
    def _():
        pltpu.sync_copy(znd.at[pl.ds(NS * ROWS_SUB, TAIL)],
                        num_s.at[pl.ds(NS * ROWS_SUB, TAIL)])

    @pl.when(s == 0)
    def _():
        pltpu.sync_copy(zn, den_s)

    pltpu.sync_copy(el, el_v)
    pltpu.sync_copy(er, er_v)

    plsc.subcore_barrier()

    def idx_fetch(ci, b):
        pltpu.async_copy(srcm.at[s, ci], srcc.at[b], isem.at[b])
        pltpu.async_copy(dstm.at[s, ci], dstc.at[b], isem.at[b])

    def idx_wait(ci, b):
        pltpu.make_async_copy(srcm.at[s, ci], srcc.at[b], isem.at[b]).wait()
        pltpu.make_async_copy(dstm.at[s, ci], dstc.at[b], isem.at[b]).wait()

    def compute_w(b):
        for j in range(CH // 16):
            sl = pl.ds(j * 16, 16)
            s16 = srcc[b, sl]
            d16 = dstc[b, sl]
            x = plsc.load_gather(el_v, [s16]) + plsc.load_gather(er_v, [d16])
            e = jnp.maximum(x, 0.2 * x)
            wc[b, sl] = jnp.exp(e)

    def num_wait(cj):
        br = lax.rem(cj, 6)
        pltpu.make_async_copy(rows_v.at[br], num_s.at[dstc.at[lax.rem(cj, 8)]],
                              ssem.at[br]).wait()

    def den_wait(cj):
        b8 = lax.rem(cj, 8)
        pltpu.make_async_copy(wc.at[b8], den_s.at[dstc.at[b8]],
                              dsem.at[b8]).wait()

    def prep(ci):
        b8 = lax.rem(ci, 8)
        idx_wait(ci, b8)
        compute_w(b8)
        pltpu.async_copy(hp.at[srcc.at[b8]], rows_v.at[lax.rem(ci, 6)],
                         gsem.at[lax.rem(ci, 6)])

    for ci in range(4):
        idx_fetch(ci, ci)
    prep(0)
    prep(1)

    def _iter(ci, carry):
        br = lax.rem(ci, 6)
        b8 = lax.rem(ci, 8)

        @pl.when(ci >= 4)
        def _():
            num_wait(ci - 4)

        @pl.when(ci + 2 < NCHUNK)
        def _():
            prep(ci + 2)

        @pl.when(ci + 4 < NCHUNK)
        def _():
            @pl.when(ci >= 4)
            def _():
                den_wait(ci - 4)

            idx_fetch(ci + 4, lax.rem(ci + 4, 8))

        pltpu.make_async_copy(hp.at[srcc.at[b8]], rows_v.at[br],
                              gsem.at[br]).wait()

        for g in range(CH // 16):
            w16 = wc[b8, pl.ds(g * 16, 16)]
            for k in range(16):
                w1 = w16[k]
                for j in range(D // 16):
                    sl = pl.ds(j * 16, 16)
                    i = g * 16 + k
                    rows_v[br, i, sl] = rows_v[br, i, sl] * w1

        pltpu.async_copy(wc.at[b8], den_s.at[dstc.at[b8]], dsem.at[b8],
                         add=True)
        pltpu.async_copy(rows_v.at[br], num_s.at[dstc.at[b8]], ssem.at[br],
                         add=True)
        return carry

    lax.fori_loop(0, NCHUNK, _iter, 0)

    for k in range(4):
        num_wait(NCHUNK - 4 + k)
    for k in range(8):
        den_wait(NCHUNK - 8 + k)

    plsc.subcore_barrier()

    pltpu.sync_copy(num_s.at[pl.ds(s * ROWS_SUB, ROWS_SUB)],
                    num_o.at[pl.ds(s * ROWS_SUB, ROWS_SUB)])

    @pl.when(s == NS - 1)
    def _():
        pltpu.sync_copy(num_s.at[pl.ds(NS * ROWS_SUB, TAIL)],
                        num_o.at[pl.ds(NS * ROWS_SUB, TAIL)])

    @pl.when(s == 0)
    def _():
        pltpu.sync_copy(den_s, den_o)


_edge = pl.kernel(
    _edge_body,
    out_type=[
        jax.ShapeDtypeStruct((N, D), jnp.float32),
        jax.ShapeDtypeStruct((N,), jnp.float32),
    ],
    mesh=plsc.VectorSubcoreMesh(core_axis_name="c", subcore_axis_name="s",
                                num_cores=1, num_subcores=NS),
    compiler_params=pltpu.CompilerParams(needs_layout_passes=False),
    scratch_types=[
        pltpu.VMEM((N,), jnp.float32),
        pltpu.VMEM((N,), jnp.float32),
        pltpu.VMEM((8, CH), jnp.int32),
        pltpu.VMEM((8, CH), jnp.int32),
        pltpu.VMEM((8, CH), jnp.float32),
        pltpu.VMEM((6, CH, D), jnp.float32),
        pltpu.VMEM_SHARED((N, D), jnp.float32),
        pltpu.VMEM_SHARED((N,), jnp.float32),
        pltpu.SemaphoreType.DMA((8,)),
        pltpu.SemaphoreType.DMA((6,)),
        pltpu.SemaphoreType.DMA((6,)),
        pltpu.SemaphoreType.DMA((8,)),
    ],
)


def kernel(h, edge_index, W_in, W1, a_l1, a_r1, W2, a_l2, a_r2, W_out):
    srcm = edge_index[0].reshape(NS, NCHUNK, CH)
    dstm = edge_index[1].reshape(NS, NCHUNK, CH)
    znd = jnp.zeros((N, D), jnp.float32)
    zn = jnp.zeros((N,), jnp.float32)

    hp1, el1, er1 = _proj(h, W_in, W1, a_l1, a_r1)
    num1, den1 = _edge(hp1, el1, er1, srcm, dstm, znd, zn)

    hp2, el2, er2 = _combine_proj(num1, den1, W2, a_l2, a_r2)
    num2, den2 = _edge(hp2, el2, er2, srcm, dstm, znd, zn)

    return _combine_out(num2, den2, W_out)
